# Initial kernel scaffold; baseline (speedup 1.0000x reference)
import jax
import jax.numpy as jnp
from jax.experimental import pallas as pl


def kernel(x, edge_index, Wl, bl, Wr, Wa, ba, Wagg, bagg, Wd, bd):
    raise NotImplementedError("write your pallas kernel here")


# speedup vs baseline: 1.2689x; 1.2689x over previous
import functools

import jax
import jax.numpy as jnp
from jax import lax
from jax.experimental import pallas as pl
from jax.experimental.pallas import tpu as pltpu
from jax.experimental.pallas import tpu_sc as plsc

N = 10000
E = 320000
H = 128
OUT = 64
ATT = 32
NGNN = 3
NCONV = 4

NC = 2
NS = 16
NW = NC * NS
K = 128
CH = 80
IG = 8
NG = CH // IG
EPAD = NW * CH * K
NACC = 10240
RPT = NACC // NS
ZR = 32
BR = 1000
NB = N // BR


def _segsum_body(g_hbm, srcw, dstw, zeros_hbm, out_hbm,
                 src_v, dst_v, rows_v, zero_v, acc_sh, sem):
    c = lax.axis_index("c")
    s = lax.axis_index("s")
    w = c * NS + s
    pltpu.sync_copy(zeros_hbm, zero_v)
    for z in range(RPT // ZR):
        pltpu.sync_copy(zero_v, acc_sh.at[pl.ds(s * RPT + z * ZR, ZR)])
    plsc.subcore_barrier()

    def group(gi, carry):
        pltpu.sync_copy(srcw.at[w, pl.ds(gi * IG, IG)], src_v)
        pltpu.sync_copy(dstw.at[w, pl.ds(gi * IG, IG)], dst_v)
        for j in range(IG):
            pltpu.async_copy(g_hbm.at[src_v.at[j]], rows_v, sem).wait()
            pltpu.sync_copy(rows_v, acc_sh.at[dst_v.at[j]], add=True)
        return carry

    lax.fori_loop(0, NG, group, 0)
    plsc.subcore_barrier()
    pltpu.sync_copy(acc_sh.at[pl.ds(s * RPT, RPT)],
                    out_hbm.at[c, pl.ds(s * RPT, RPT)])


@functools.cache
def _segsum_fn():
    return pl.kernel(
        _segsum_body,
        out_type=jax.ShapeDtypeStruct((NC, NACC, H), jnp.float32),
        mesh=plsc.VectorSubcoreMesh(core_axis_name="c", subcore_axis_name="s"),
        scratch_types=[
            pltpu.VMEM((IG, K), jnp.int32),
            pltpu.VMEM((IG, K), jnp.int32),
            pltpu.VMEM((K, H), jnp.float32),
            pltpu.VMEM((ZR, H), jnp.float32),
            pltpu.VMEM_SHARED((NACC, H), jnp.float32),
            pltpu.SemaphoreType.DMA,
        ],
    )


def _segsum_sc(g, srcw, dstw, zeros_zr):
    return _segsum_fn()(g, srcw, dstw, zeros_zr)


def _first_body(x_ref, wl_ref, wr_ref, bl_ref, g_out, r_out):
    h = x_ref[...]
    g_out[...] = jnp.dot(h, wl_ref[...], preferred_element_type=jnp.float32)
    r_out[...] = (jnp.dot(h, wr_ref[...], preferred_element_type=jnp.float32)
                  + bl_ref[...])


def _first_tc(x, wl, wr, blv):
    return pl.pallas_call(
        _first_body,
        grid=(NB,),
        in_specs=[
            pl.BlockSpec((BR, H), lambda i: (i, 0)),
            pl.BlockSpec((H, H), lambda i: (0, 0)),
            pl.BlockSpec((H, H), lambda i: (0, 0)),
            pl.BlockSpec((1, H), lambda i: (0, 0)),
        ],
        out_specs=[pl.BlockSpec((BR, H), lambda i: (i, 0)),
                   pl.BlockSpec((BR, H), lambda i: (i, 0))],
        out_shape=[jax.ShapeDtypeStruct((N, H), jnp.float32)] * 2,
    )(x, wl, wr, blv)


def _mid_body(s_ref, r_ref, d0_ref, d1_ref, wl_ref, wr_ref, bl_ref,
              g_out, r_out):
    d = jnp.maximum(d0_ref[...] + d1_ref[...], 1.0)
    h = (s_ref[0] + s_ref[1]) / d + r_ref[...]
    h = jnp.where(h > 0, h, 0.1 * h)
    g_out[...] = jnp.dot(h, wl_ref[...], preferred_element_type=jnp.float32)
    r_out[...] = (jnp.dot(h, wr_ref[...], preferred_element_type=jnp.float32)
                  + bl_ref[...])


def _mid_tc(s_pair, r_prev, d0, d1, wl, wr, blv):
    return pl.pallas_call(
        _mid_body,
        grid=(NB,),
        in_specs=[
            pl.BlockSpec((NC, BR, H), lambda i: (0, i, 0)),
            pl.BlockSpec((BR, H), lambda i: (i, 0)),
            pl.BlockSpec((BR, 1), lambda i: (i, 0)),
            pl.BlockSpec((BR, 1), lambda i: (i, 0)),
            pl.BlockSpec((H, H), lambda i: (0, 0)),
            pl.BlockSpec((H, H), lambda i: (0, 0)),
            pl.BlockSpec((1, H), lambda i: (0, 0)),
        ],
        out_specs=[pl.BlockSpec((BR, H), lambda i: (i, 0)),
                   pl.BlockSpec((BR, H), lambda i: (i, 0))],
        out_shape=[jax.ShapeDtypeStruct((N, H), jnp.float32)] * 2,
    )(s_pair, r_prev, d0, d1, wl, wr, blv)


def _att_body(s0_ref, r0_ref, s1_ref, r1_ref, s2_ref, r2_ref,
              d0_ref, d1_ref, wa_ref, ba_ref, wagg_ref, bagg_ref,
              wd_ref, bd_ref, out_ref):
    d = jnp.maximum(d0_ref[...] + d1_ref[...], 1.0)
    s_refs = (s0_ref, s1_ref, s2_ref)
    r_refs = (r0_ref, r1_ref, r2_ref)
    preds = []
    ts = []
    for i in range(NGNN):
        p = (s_refs[i][0] + s_refs[i][1]) / d + r_refs[i][...]
        preds.append(p)
        t = jnp.dot(p, wa_ref[i], preferred_element_type=jnp.float32)
        ts.append(t + ba_ref[i][None, :])
    cols = []
    for k in range(NGNN):
        a = bagg_ref[0, k]
        for i in range(NGNN):
            a = a + jnp.dot(ts[i], wagg_ref[i * ATT:(i + 1) * ATT, k:k + 1],
                            preferred_element_type=jnp.float32)
        cols.append(a)
    m = (cols[0] + cols[1] + cols[2]) * (1.0 / NGNN)
    cols = [ak - m for ak in cols]
    amin = jnp.minimum(jnp.minimum(cols[0], cols[1]), cols[2])
    amax = jnp.maximum(jnp.maximum(cols[0], cols[1]), cols[2])
    inv = 1.0 / (amax - amin)
    final = (cols[0] - amin) * inv * preds[0]
    final = final + (1.0 / NGNN) * preds[0]
    for i in range(1, NGNN):
        final = final + ((cols[i] - amin) * inv + 1.0 / NGNN) * preds[i]
    out_ref[...] = (jnp.dot(final, wd_ref[...],
                            preferred_element_type=jnp.float32)
                    + bd_ref[...])


def _att_tc(s_list, r_list, d0, d1, Wa, ba, Wagg, bagg, Wd, bd):
    args = []
    specs = []
    for i in range(NGNN):
        args += [s_list[i], r_list[i]]
        specs += [pl.BlockSpec((NC, BR, H), lambda i: (0, i, 0)),
                  pl.BlockSpec((BR, H), lambda i: (i, 0))]
    args += [d0, d1, Wa, ba, Wagg, bagg.reshape(1, NGNN), Wd,
             bd.reshape(1, OUT)]
    specs += [
        pl.BlockSpec((BR, 1), lambda i: (i, 0)),
        pl.BlockSpec((BR, 1), lambda i: (i, 0)),
        pl.BlockSpec((NGNN, H, ATT), lambda i: (0, 0, 0)),
        pl.BlockSpec((NGNN, ATT), lambda i: (0, 0)),
        pl.BlockSpec((NGNN * ATT, NGNN), lambda i: (0, 0)),
        pl.BlockSpec((1, NGNN), lambda i: (0, 0)),
        pl.BlockSpec((H, OUT), lambda i: (0, 0)),
        pl.BlockSpec((1, OUT), lambda i: (0, 0)),
    ]
    return pl.pallas_call(
        _att_body,
        grid=(NB,),
        in_specs=specs,
        out_specs=pl.BlockSpec((BR, OUT), lambda i: (i, 0)),
        out_shape=jax.ShapeDtypeStruct((N, OUT), jnp.float32),
    )(*args)


def kernel(x, edge_index, Wl, bl, Wr, Wa, ba, Wagg, bagg, Wd, bd):
    src = edge_index[0].astype(jnp.int32)
    dst = edge_index[1].astype(jnp.int32)
    pad = EPAD - E
    srcw = jnp.concatenate([src, jnp.zeros((pad,), jnp.int32)]).reshape(
        NW, CH, K)
    dstw = jnp.concatenate([dst, jnp.full((pad,), N, jnp.int32)]).reshape(
        NW, CH, K)
    zeros_zr = jnp.zeros((ZR, H), jnp.float32)

    degp = _segsum_sc(jnp.ones((N, H), jnp.float32), srcw, dstw, zeros_zr)
    d0 = degp[0, :N, 0:1]
    d1 = degp[1, :N, 0:1]

    s_last = []
    r_last = []
    for g in range(NGNN):
        Gg, Rg = _first_tc(x, Wl[g, 0], Wr[g, 0], bl[g, 0].reshape(1, H))
        for j in range(1, NCONV):
            Sg = _segsum_sc(Gg, srcw, dstw, zeros_zr)
            Gg, Rg = _mid_tc(Sg, Rg, d0, d1, Wl[g, j], Wr[g, j],
                             bl[g, j].reshape(1, H))
        Sg = _segsum_sc(Gg, srcw, dstw, zeros_zr)
        s_last.append(Sg)
        r_last.append(Rg)

    return _att_tc(s_last, r_last, d0, d1, Wa, ba, Wagg, bagg, Wd, bd)


# speedup vs baseline: 1.4289x; 1.1261x over previous
import functools

import jax
import jax.numpy as jnp
from jax import lax
from jax.experimental import pallas as pl
from jax.experimental.pallas import tpu as pltpu
from jax.experimental.pallas import tpu_sc as plsc

N = 10000
E = 320000
H = 128
OUT = 64
ATT = 32
NGNN = 3
NCONV = 4

NC = 2
NS = 16
NW = NC * NS
K = 80
CH = 128
NBUF = 4
QC = 32
NGQ = QC // NBUF
NQ = CH // QC
EPAD = NW * CH * K
NACC = 10240
RPT = NACC // NS
BR = 1000
NB = N // BR


def _segsum_body(g_hbm, srcw, dstw, zeros_hbm, out_hbm,
                 src_v, dst_v, b0, b1, b2, b3, acc_sh,
                 sg0, sg1, sg2, sg3, ss0, ss1, ss2, ss3):
    c = lax.axis_index("c")
    s = lax.axis_index("s")
    w = c * NS + s
    bufs = (b0, b1, b2, b3)
    sgs = (sg0, sg1, sg2, sg3)
    sss = (ss0, ss1, ss2, ss3)
    pltpu.sync_copy(zeros_hbm, b0)
    for z in range(RPT // K):
        pltpu.sync_copy(b0, acc_sh.at[pl.ds(s * RPT + z * K, K)])
    plsc.subcore_barrier()

    def quarter(q, carry):
        pltpu.sync_copy(srcw.at[w, pl.ds(q * QC, QC)], src_v)
        pltpu.sync_copy(dstw.at[w, pl.ds(q * QC, QC)], dst_v)

        def group(t, carry2):
            gg = q * NGQ + t
            for b in range(NBUF):
                ch = t * NBUF + b

                @pl.when(gg > 0)
                def _():
                    pltpu.make_async_copy(zeros_hbm, bufs[b], sss[b]).wait()

                pltpu.make_async_copy(
                    g_hbm.at[src_v.at[ch]], bufs[b], sgs[b]).start()
            for b in range(NBUF):
                ch = t * NBUF + b
                pltpu.make_async_copy(
                    g_hbm.at[src_v.at[ch]], bufs[b], sgs[b]).wait()
                pltpu.async_copy(
                    bufs[b], acc_sh.at[dst_v.at[ch]], sss[b], add=True)
            return carry2

        lax.fori_loop(0, NGQ, group, 0)
        return carry

    lax.fori_loop(0, NQ, quarter, 0)
    for b in range(NBUF):
        pltpu.make_async_copy(zeros_hbm, bufs[b], sss[b]).wait()
    plsc.subcore_barrier()
    pltpu.sync_copy(acc_sh.at[pl.ds(s * RPT, RPT)],
                    out_hbm.at[c, pl.ds(s * RPT, RPT)])


@functools.cache
def _segsum_fn():
    return pl.kernel(
        _segsum_body,
        out_type=jax.ShapeDtypeStruct((NC, NACC, H), jnp.float32),
        mesh=plsc.VectorSubcoreMesh(core_axis_name="c", subcore_axis_name="s"),
        scratch_types=[
            pltpu.VMEM((QC, K), jnp.int32),
            pltpu.VMEM((QC, K), jnp.int32),
            pltpu.VMEM((K, H), jnp.float32),
            pltpu.VMEM((K, H), jnp.float32),
            pltpu.VMEM((K, H), jnp.float32),
            pltpu.VMEM((K, H), jnp.float32),
            pltpu.VMEM_SHARED((NACC, H), jnp.float32),
            pltpu.SemaphoreType.DMA,
            pltpu.SemaphoreType.DMA,
            pltpu.SemaphoreType.DMA,
            pltpu.SemaphoreType.DMA,
            pltpu.SemaphoreType.DMA,
            pltpu.SemaphoreType.DMA,
            pltpu.SemaphoreType.DMA,
            pltpu.SemaphoreType.DMA,
        ],
    )


def _segsum_sc(g, srcw, dstw, zeros_zr):
    return _segsum_fn()(g, srcw, dstw, zeros_zr)


def _first_body(x_ref, wl_ref, wr_ref, bl_ref, g_out, r_out):
    h = x_ref[...]
    g_out[...] = jnp.dot(h, wl_ref[...], preferred_element_type=jnp.float32)
    r_out[...] = (jnp.dot(h, wr_ref[...], preferred_element_type=jnp.float32)
                  + bl_ref[...])


def _first_tc(x, wl, wr, blv):
    return pl.pallas_call(
        _first_body,
        grid=(NB,),
        in_specs=[
            pl.BlockSpec((BR, H), lambda i: (i, 0)),
            pl.BlockSpec((H, H), lambda i: (0, 0)),
            pl.BlockSpec((H, H), lambda i: (0, 0)),
            pl.BlockSpec((1, H), lambda i: (0, 0)),
        ],
        out_specs=[pl.BlockSpec((BR, H), lambda i: (i, 0)),
                   pl.BlockSpec((BR, H), lambda i: (i, 0))],
        out_shape=[jax.ShapeDtypeStruct((N, H), jnp.float32)] * 2,
    )(x, wl, wr, blv)


def _mid_body(s_ref, r_ref, d0_ref, d1_ref, wl_ref, wr_ref, bl_ref,
              g_out, r_out):
    d = jnp.maximum(d0_ref[...] + d1_ref[...], 1.0)
    h = (s_ref[0] + s_ref[1]) / d + r_ref[...]
    h = jnp.where(h > 0, h, 0.1 * h)
    g_out[...] = jnp.dot(h, wl_ref[...], preferred_element_type=jnp.float32)
    r_out[...] = (jnp.dot(h, wr_ref[...], preferred_element_type=jnp.float32)
                  + bl_ref[...])


def _mid_tc(s_pair, r_prev, d0, d1, wl, wr, blv):
    return pl.pallas_call(
        _mid_body,
        grid=(NB,),
        in_specs=[
            pl.BlockSpec((NC, BR, H), lambda i: (0, i, 0)),
            pl.BlockSpec((BR, H), lambda i: (i, 0)),
            pl.BlockSpec((BR, 1), lambda i: (i, 0)),
            pl.BlockSpec((BR, 1), lambda i: (i, 0)),
            pl.BlockSpec((H, H), lambda i: (0, 0)),
            pl.BlockSpec((H, H), lambda i: (0, 0)),
            pl.BlockSpec((1, H), lambda i: (0, 0)),
        ],
        out_specs=[pl.BlockSpec((BR, H), lambda i: (i, 0)),
                   pl.BlockSpec((BR, H), lambda i: (i, 0))],
        out_shape=[jax.ShapeDtypeStruct((N, H), jnp.float32)] * 2,
    )(s_pair, r_prev, d0, d1, wl, wr, blv)


def _att_body(s0_ref, r0_ref, s1_ref, r1_ref, s2_ref, r2_ref,
              d0_ref, d1_ref, wa_ref, ba_ref, wagg_ref, bagg_ref,
              wd_ref, bd_ref, out_ref):
    d = jnp.maximum(d0_ref[...] + d1_ref[...], 1.0)
    s_refs = (s0_ref, s1_ref, s2_ref)
    r_refs = (r0_ref, r1_ref, r2_ref)
    preds = []
    ts = []
    for i in range(NGNN):
        p = (s_refs[i][0] + s_refs[i][1]) / d + r_refs[i][...]
        preds.append(p)
        t = jnp.dot(p, wa_ref[i], preferred_element_type=jnp.float32)
        ts.append(t + ba_ref[i][None, :])
    cols = []
    for k in range(NGNN):
        a = bagg_ref[0, k]
        for i in range(NGNN):
            a = a + jnp.dot(ts[i], wagg_ref[i * ATT:(i + 1) * ATT, k:k + 1],
                            preferred_element_type=jnp.float32)
        cols.append(a)
    m = (cols[0] + cols[1] + cols[2]) * (1.0 / NGNN)
    cols = [ak - m for ak in cols]
    amin = jnp.minimum(jnp.minimum(cols[0], cols[1]), cols[2])
    amax = jnp.maximum(jnp.maximum(cols[0], cols[1]), cols[2])
    inv = 1.0 / (amax - amin)
    final = (cols[0] - amin) * inv * preds[0]
    final = final + (1.0 / NGNN) * preds[0]
    for i in range(1, NGNN):
        final = final + ((cols[i] - amin) * inv + 1.0 / NGNN) * preds[i]
    out_ref[...] = (jnp.dot(final, wd_ref[...],
                            preferred_element_type=jnp.float32)
                    + bd_ref[...])


def _att_tc(s_list, r_list, d0, d1, Wa, ba, Wagg, bagg, Wd, bd):
    args = []
    specs = []
    for i in range(NGNN):
        args += [s_list[i], r_list[i]]
        specs += [pl.BlockSpec((NC, BR, H), lambda i: (0, i, 0)),
                  pl.BlockSpec((BR, H), lambda i: (i, 0))]
    args += [d0, d1, Wa, ba, Wagg, bagg.reshape(1, NGNN), Wd,
             bd.reshape(1, OUT)]
    specs += [
        pl.BlockSpec((BR, 1), lambda i: (i, 0)),
        pl.BlockSpec((BR, 1), lambda i: (i, 0)),
        pl.BlockSpec((NGNN, H, ATT), lambda i: (0, 0, 0)),
        pl.BlockSpec((NGNN, ATT), lambda i: (0, 0)),
        pl.BlockSpec((NGNN * ATT, NGNN), lambda i: (0, 0)),
        pl.BlockSpec((1, NGNN), lambda i: (0, 0)),
        pl.BlockSpec((H, OUT), lambda i: (0, 0)),
        pl.BlockSpec((1, OUT), lambda i: (0, 0)),
    ]
    return pl.pallas_call(
        _att_body,
        grid=(NB,),
        in_specs=specs,
        out_specs=pl.BlockSpec((BR, OUT), lambda i: (i, 0)),
        out_shape=jax.ShapeDtypeStruct((N, OUT), jnp.float32),
    )(*args)


def kernel(x, edge_index, Wl, bl, Wr, Wa, ba, Wagg, bagg, Wd, bd):
    src = edge_index[0].astype(jnp.int32)
    dst = edge_index[1].astype(jnp.int32)
    pad = EPAD - E
    srcw = jnp.concatenate([src, jnp.zeros((pad,), jnp.int32)]).reshape(
        NW, CH, K)
    dstw = jnp.concatenate([dst, jnp.full((pad,), N, jnp.int32)]).reshape(
        NW, CH, K)
    zeros_zr = jnp.zeros((K, H), jnp.float32)

    degp = _segsum_sc(jnp.ones((N, H), jnp.float32), srcw, dstw, zeros_zr)
    d0 = degp[0, :N, 0:1]
    d1 = degp[1, :N, 0:1]

    s_last = []
    r_last = []
    for g in range(NGNN):
        Gg, Rg = _first_tc(x, Wl[g, 0], Wr[g, 0], bl[g, 0].reshape(1, H))
        for j in range(1, NCONV):
            Sg = _segsum_sc(Gg, srcw, dstw, zeros_zr)
            Gg, Rg = _mid_tc(Sg, Rg, d0, d1, Wl[g, j], Wr[g, j],
                             bl[g, j].reshape(1, H))
        Sg = _segsum_sc(Gg, srcw, dstw, zeros_zr)
        s_last.append(Sg)
        r_last.append(Rg)

    return _att_tc(s_last, r_last, d0, d1, Wa, ba, Wagg, bagg, Wd, bd)
